# baseline (speedup 1.0000x reference)
import functools

import jax
import jax.numpy as jnp
from jax import lax
from jax.experimental import pallas as pl
from jax.experimental.pallas import tpu as pltpu
from jax.experimental.pallas import tpu_sc as plsc

_N, _E, _D, _DE = 10000, 320000, 128, 16
_NC, _NS = 2, 16
_NW = _NC * _NS
_EPW = _E // _NW
_BLK = 80
_NBLK = _EPW // _BLK
_RPS = 624
_RTAIL = _N - _NS * _RPS
_G = 5
_GE = _G * _BLK
_NG = _EPW // _GE


def _pq_body(x_ref, w_ref, p_ref, q_ref):
    pq = jnp.dot(x_ref[...], w_ref[...], preferred_element_type=jnp.float32)
    p_ref[...] = pq[:, :_DE]
    q_ref[...] = pq[:, _DE:]


def _r_body(ea_ref, w_ref, b_ref, r_ref):
    r_ref[...] = (
        jnp.dot(ea_ref[...], w_ref[...], preferred_element_type=jnp.float32)
        + b_ref[...]
    )


_sc_mesh = plsc.VectorSubcoreMesh(core_axis_name="c", subcore_axis_name="s")


@functools.partial(
    pl.kernel,
    out_type=(
        jax.ShapeDtypeStruct((_E, _DE), jnp.float32),
        jax.ShapeDtypeStruct((_NC, _N, _DE), jnp.float32),
    ),
    mesh=_sc_mesh,
    scratch_types=[
        pltpu.VMEM((_NBLK, _BLK), jnp.int32),
        pltpu.VMEM((_NBLK, _BLK), jnp.int32),
        pltpu.VMEM((_G, _BLK, _DE), jnp.float32),
        pltpu.VMEM((_G, _BLK, _DE), jnp.float32),
        pltpu.VMEM((_G, _BLK, _DE), jnp.float32),
        pltpu.VMEM((_G, _BLK, _DE), jnp.float32),
        pltpu.VMEM((_GE, _DE), jnp.float32),
        pltpu.VMEM_SHARED((_N, _DE), jnp.float32),
        pltpu.SemaphoreType.DMA,
        pltpu.SemaphoreType.DMA,
    ],
    compiler_params=pltpu.CompilerParams(use_tc_tiling_on_sc=False),
)
def _sc_edge(p_hbm, q_hbm, r_hbm, row_hbm, col_hbm, z_hbm,
             out_hbm, agg_hbm,
             rowidx_v, colidx_v, p_v, q_v, r_v, o_v, of_v, agg_sp,
             sem_in, sem_out):
    cid = lax.axis_index("c")
    sid = lax.axis_index("s")
    wid = cid * _NS + sid
    ebase = wid * _EPW

    pltpu.sync_copy(z_hbm.at[pl.ds(0, _RPS)], agg_sp.at[pl.ds(sid * _RPS, _RPS)])

    @pl.when(sid == 0)
    def _zero_tail():
        pltpu.sync_copy(z_hbm.at[pl.ds(0, _RTAIL)],
                        agg_sp.at[pl.ds(_NS * _RPS, _RTAIL)])

    pltpu.sync_copy(row_hbm.at[wid], rowidx_v)
    pltpu.sync_copy(col_hbm.at[wid], colidx_v)

    plsc.subcore_barrier()

    blk0 = wid * _NBLK

    def it(g, carry):
        ds = []
        for b in range(_G):
            i = g * _G + b
            ds.append(pltpu.async_copy(
                p_hbm.at[rowidx_v.at[i]], p_v.at[b], sem_in))
            ds.append(pltpu.async_copy(
                q_hbm.at[colidx_v.at[i]], q_v.at[b], sem_in))
        ds.append(pltpu.async_copy(
            r_hbm.at[pl.ds(blk0 + g * _G, _G)], r_v, sem_in))
        for d in ds:
            d.wait()

        def body(j, c):
            for b in range(_G):
                o = jnp.maximum(p_v[b, j, :] + q_v[b, j, :] + r_v[b, j, :],
                                0.0)
                o_v[b, j, :] = o
                of_v[b * _BLK + j, :] = o
            return c

        lax.fori_loop(0, _BLK, body, 0, unroll=2)

        pltpu.sync_copy(of_v, out_hbm.at[pl.ds(ebase + g * _GE, _GE)])
        for b in range(_G):
            i = g * _G + b
            pltpu.sync_copy(o_v.at[b], agg_sp.at[colidx_v.at[i]], add=True)
        return carry

    lax.fori_loop(0, _NG, it, 0)

    plsc.subcore_barrier()
    pltpu.sync_copy(
        agg_sp.at[pl.ds(sid * _RPS, _RPS)],
        agg_hbm.at[cid].at[pl.ds(sid * _RPS, _RPS)],
    )

    @pl.when(sid == 0)
    def _write_tail():
        pltpu.sync_copy(
            agg_sp.at[pl.ds(_NS * _RPS, _RTAIL)],
            agg_hbm.at[cid].at[pl.ds(_NS * _RPS, _RTAIL)],
        )


def _node_body(x_ref, a0_ref, a1_ref, wx_ref, wa_ref, b_ref, o_ref):
    acc = jnp.dot(x_ref[...], wx_ref[...], preferred_element_type=jnp.float32)
    acc += jnp.dot(a0_ref[...] + a1_ref[...], wa_ref[...],
                   preferred_element_type=jnp.float32)
    o_ref[...] = jnp.maximum(acc + b_ref[...], 0.0)


def kernel(x, edge_index, edge_attr, W_edge, b_edge, W_node, b_node):
    Ws = W_edge[:_D]
    Wd = W_edge[_D:2 * _D]
    Wa = W_edge[2 * _D:]
    W_sd = jnp.concatenate([Ws, Wd], axis=1)

    row = edge_index[0].reshape(_NW, _NBLK, _BLK)
    col = edge_index[1].reshape(_NW, _NBLK, _BLK)

    nb = 5
    P, Q = pl.pallas_call(
        _pq_body,
        grid=(nb,),
        in_specs=[
            pl.BlockSpec((_N // nb, _D), lambda i: (i, 0)),
            pl.BlockSpec((_D, 2 * _DE), lambda i: (0, 0)),
        ],
        out_specs=[
            pl.BlockSpec((_N // nb, _DE), lambda i: (i, 0)),
            pl.BlockSpec((_N // nb, _DE), lambda i: (i, 0)),
        ],
        out_shape=[jax.ShapeDtypeStruct((_N, _DE), jnp.float32)] * 2,
    )(x, W_sd)

    rb = 40
    R = pl.pallas_call(
        _r_body,
        grid=(rb,),
        in_specs=[
            pl.BlockSpec((_E // rb, _DE), lambda i: (i, 0)),
            pl.BlockSpec((_DE, _DE), lambda i: (0, 0)),
            pl.BlockSpec((1, _DE), lambda i: (0, 0)),
        ],
        out_specs=pl.BlockSpec((_E // rb, _DE), lambda i: (i, 0)),
        out_shape=jax.ShapeDtypeStruct((_E, _DE), jnp.float32),
    )(edge_attr, Wa, b_edge.reshape(1, _DE))

    zeros = jnp.zeros((_RPS, _DE), jnp.float32)
    edge_attr_new, agg2 = _sc_edge(
        P, Q, R.reshape(_NW * _NBLK, _BLK, _DE), row, col, zeros)

    xb = 5
    x_new = pl.pallas_call(
        _node_body,
        grid=(xb,),
        in_specs=[
            pl.BlockSpec((_N // xb, _D), lambda i: (i, 0)),
            pl.BlockSpec((_N // xb, _DE), lambda i: (i, 0)),
            pl.BlockSpec((_N // xb, _DE), lambda i: (i, 0)),
            pl.BlockSpec((_D, _D), lambda i: (0, 0)),
            pl.BlockSpec((_DE, _D), lambda i: (0, 0)),
            pl.BlockSpec((1, _D), lambda i: (0, 0)),
        ],
        out_specs=pl.BlockSpec((_N // xb, _D), lambda i: (i, 0)),
        out_shape=jax.ShapeDtypeStruct((_N, _D), jnp.float32),
    )(x, agg2[0], agg2[1], W_node[:_D], W_node[_D:], b_node.reshape(1, _D))

    return (x_new, edge_attr_new)


# speedup vs baseline: 1.5064x; 1.5064x over previous
import functools

import jax
import jax.numpy as jnp
from jax import lax
from jax.experimental import pallas as pl
from jax.experimental.pallas import tpu as pltpu
from jax.experimental.pallas import tpu_sc as plsc

_N, _E, _D, _DE = 10000, 320000, 128, 16
_NC, _NS = 2, 16
_NW = _NC * _NS
_EPW = _E // _NW
_BLK = 80
_NBLK = _EPW // _BLK
_RPS = 624
_RTAIL = _N - _NS * _RPS
_G = 5
_GE = _G * _BLK
_NG = _EPW // _GE


def _pq_body(x_ref, w_ref, p_ref, q_ref):
    pq = jnp.dot(x_ref[...], w_ref[...], preferred_element_type=jnp.float32)
    p_ref[...] = pq[:, :_DE]
    q_ref[...] = pq[:, _DE:]


def _r_body(ea_ref, w_ref, b_ref, r_ref):
    r_ref[...] = (
        jnp.dot(ea_ref[...], w_ref[...], preferred_element_type=jnp.float32)
        + b_ref[...]
    )


_sc_mesh = plsc.VectorSubcoreMesh(core_axis_name="c", subcore_axis_name="s")


@functools.partial(
    pl.kernel,
    out_type=(
        jax.ShapeDtypeStruct((_E // 8, _D), jnp.float32),
        jax.ShapeDtypeStruct((_NC, _N, _DE), jnp.float32),
    ),
    mesh=_sc_mesh,
    scratch_types=[
        pltpu.VMEM((_NBLK, _BLK), jnp.int32),
        pltpu.VMEM((_NBLK, _BLK), jnp.int32),
        pltpu.VMEM((_G, _BLK, _DE), jnp.float32),
        pltpu.VMEM((_G, _BLK, _DE), jnp.float32),
        pltpu.VMEM((_G, _BLK, _DE), jnp.float32),
        pltpu.VMEM((_G, _BLK, _DE), jnp.float32),
        pltpu.VMEM((_GE // 8, _D), jnp.float32),
        pltpu.VMEM_SHARED((_N, _DE), jnp.float32),
        pltpu.SemaphoreType.DMA,
        pltpu.SemaphoreType.DMA,
    ],
    compiler_params=pltpu.CompilerParams(use_tc_tiling_on_sc=False),
)
def _sc_edge(p_hbm, q_hbm, r_hbm, row_hbm, col_hbm, z_hbm,
             out_hbm, agg_hbm,
             rowidx_v, colidx_v, p_v, q_v, r_v, o_v, of_v, agg_sp,
             sem_in, sem_out):
    cid = lax.axis_index("c")
    sid = lax.axis_index("s")
    wid = cid * _NS + sid
    ebase = wid * _EPW

    pltpu.sync_copy(z_hbm.at[pl.ds(0, _RPS)], agg_sp.at[pl.ds(sid * _RPS, _RPS)])

    @pl.when(sid == 0)
    def _zero_tail():
        pltpu.sync_copy(z_hbm.at[pl.ds(0, _RTAIL)],
                        agg_sp.at[pl.ds(_NS * _RPS, _RTAIL)])

    pltpu.sync_copy(row_hbm.at[wid], rowidx_v)
    pltpu.sync_copy(col_hbm.at[wid], colidx_v)

    plsc.subcore_barrier()

    blk0 = wid * _NBLK

    def it(g, carry):
        ds = []
        for b in range(_G):
            i = g * _G + b
            ds.append(pltpu.async_copy(
                p_hbm.at[rowidx_v.at[i]], p_v.at[b], sem_in))
            ds.append(pltpu.async_copy(
                q_hbm.at[colidx_v.at[i]], q_v.at[b], sem_in))
        ds.append(pltpu.async_copy(
            r_hbm.at[pl.ds(blk0 + g * _G, _G)], r_v, sem_in))
        for d in ds:
            d.wait()

        def body(jr, c):
            for b in range(_G):
                for cc in range(8):
                    j = jr * 8 + cc
                    o = jnp.maximum(
                        p_v[b, j, :] + q_v[b, j, :] + r_v[b, j, :], 0.0)
                    o_v[b, j, :] = o
                    of_v[b * (_BLK // 8) + jr, pl.ds(cc * _DE, _DE)] = o
            return c

        lax.fori_loop(0, _BLK // 8, body, 0)

        pltpu.sync_copy(of_v,
                        out_hbm.at[pl.ds((ebase + g * _GE) // 8, _GE // 8)])
        for b in range(_G):
            i = g * _G + b
            pltpu.sync_copy(o_v.at[b], agg_sp.at[colidx_v.at[i]], add=True)
        return carry

    lax.fori_loop(0, _NG, it, 0)

    plsc.subcore_barrier()
    pltpu.sync_copy(
        agg_sp.at[pl.ds(sid * _RPS, _RPS)],
        agg_hbm.at[cid].at[pl.ds(sid * _RPS, _RPS)],
    )

    @pl.when(sid == 0)
    def _write_tail():
        pltpu.sync_copy(
            agg_sp.at[pl.ds(_NS * _RPS, _RTAIL)],
            agg_hbm.at[cid].at[pl.ds(_NS * _RPS, _RTAIL)],
        )


def _node_body(x_ref, a0_ref, a1_ref, wx_ref, wa_ref, b_ref, o_ref):
    acc = jnp.dot(x_ref[...], wx_ref[...], preferred_element_type=jnp.float32)
    acc += jnp.dot(a0_ref[...] + a1_ref[...], wa_ref[...],
                   preferred_element_type=jnp.float32)
    o_ref[...] = jnp.maximum(acc + b_ref[...], 0.0)


def kernel(x, edge_index, edge_attr, W_edge, b_edge, W_node, b_node):
    Ws = W_edge[:_D]
    Wd = W_edge[_D:2 * _D]
    Wa = W_edge[2 * _D:]
    W_sd = jnp.concatenate([Ws, Wd], axis=1)
    W_blk = jax.scipy.linalg.block_diag(*([Wa] * 8))
    b8 = jnp.tile(b_edge, 8).reshape(1, _D)

    row = edge_index[0].reshape(_NW, _NBLK, _BLK)
    col = edge_index[1].reshape(_NW, _NBLK, _BLK)

    nb = 5
    P, Q = pl.pallas_call(
        _pq_body,
        grid=(nb,),
        in_specs=[
            pl.BlockSpec((_N // nb, _D), lambda i: (i, 0)),
            pl.BlockSpec((_D, 2 * _DE), lambda i: (0, 0)),
        ],
        out_specs=[
            pl.BlockSpec((_N // nb, _DE), lambda i: (i, 0)),
            pl.BlockSpec((_N // nb, _DE), lambda i: (i, 0)),
        ],
        out_shape=[jax.ShapeDtypeStruct((_N, _DE), jnp.float32)] * 2,
    )(x, W_sd)

    er, rb = _E // 8, 10
    R = pl.pallas_call(
        _r_body,
        grid=(rb,),
        in_specs=[
            pl.BlockSpec((er // rb, _D), lambda i: (i, 0)),
            pl.BlockSpec((_D, _D), lambda i: (0, 0)),
            pl.BlockSpec((1, _D), lambda i: (0, 0)),
        ],
        out_specs=pl.BlockSpec((er // rb, _D), lambda i: (i, 0)),
        out_shape=jax.ShapeDtypeStruct((er, _D), jnp.float32),
    )(edge_attr.reshape(er, _D), W_blk, b8)

    zeros = jnp.zeros((_RPS, _DE), jnp.float32)
    edge_attr_new, agg2 = _sc_edge(
        P, Q, R.reshape(_NW * _NBLK, _BLK, _DE), row, col, zeros)
    edge_attr_new = edge_attr_new.reshape(_E, _DE)

    xb = 5
    x_new = pl.pallas_call(
        _node_body,
        grid=(xb,),
        in_specs=[
            pl.BlockSpec((_N // xb, _D), lambda i: (i, 0)),
            pl.BlockSpec((_N // xb, _DE), lambda i: (i, 0)),
            pl.BlockSpec((_N // xb, _DE), lambda i: (i, 0)),
            pl.BlockSpec((_D, _D), lambda i: (0, 0)),
            pl.BlockSpec((_DE, _D), lambda i: (0, 0)),
            pl.BlockSpec((1, _D), lambda i: (0, 0)),
        ],
        out_specs=pl.BlockSpec((_N // xb, _D), lambda i: (i, 0)),
        out_shape=jax.ShapeDtypeStruct((_N, _D), jnp.float32),
    )(x, agg2[0], agg2[1], W_node[:_D], W_node[_D:], b_node.reshape(1, _D))

    return (x_new, edge_attr_new)


# speedup vs baseline: 1.6870x; 1.1198x over previous
import functools

import jax
import jax.numpy as jnp
from jax import lax
from jax.experimental import pallas as pl
from jax.experimental.pallas import tpu as pltpu
from jax.experimental.pallas import tpu_sc as plsc

_N, _E, _D, _DE = 10000, 320000, 128, 16
_NC, _NS = 2, 16
_NW = _NC * _NS
_EPW = _E // _NW
_BLK = 80
_NBLK = _EPW // _BLK
_RPS = 624
_RTAIL = _N - _NS * _RPS
_G = 5
_GE = _G * _BLK
_NG = _EPW // _GE


def _pq_body(x_ref, w_ref, p_ref, q_ref):
    pq = jnp.dot(x_ref[...], w_ref[...], preferred_element_type=jnp.float32)
    p_ref[...] = pq[:, :_DE]
    q_ref[...] = pq[:, _DE:]


def _r_body(ea_ref, w_ref, b_ref, r_ref):
    r_ref[...] = (
        jnp.dot(ea_ref[...], w_ref[...], preferred_element_type=jnp.float32)
        + b_ref[...]
    )


_sc_mesh = plsc.VectorSubcoreMesh(core_axis_name="c", subcore_axis_name="s")


@functools.partial(
    pl.kernel,
    out_type=(
        jax.ShapeDtypeStruct((_E // 8, _D), jnp.float32),
        jax.ShapeDtypeStruct((_NC, _N, _DE), jnp.float32),
    ),
    mesh=_sc_mesh,
    scratch_types=[
        pltpu.VMEM((_NBLK, _BLK), jnp.int32),
        pltpu.VMEM((_NBLK, _BLK), jnp.int32),
        pltpu.VMEM((2, _G, _BLK, _DE), jnp.float32),
        pltpu.VMEM((2, _G, _BLK, _DE), jnp.float32),
        pltpu.VMEM((2, _G, _BLK, _DE), jnp.float32),
        pltpu.VMEM((_G, _BLK, _DE), jnp.float32),
        pltpu.VMEM((_GE // 8, _D), jnp.float32),
        pltpu.VMEM_SHARED((_N, _DE), jnp.float32),
        pltpu.SemaphoreType.DMA,
        pltpu.SemaphoreType.DMA,
    ],
    compiler_params=pltpu.CompilerParams(use_tc_tiling_on_sc=False),
)
def _sc_edge(p_hbm, q_hbm, r_hbm, ei_hbm, z_hbm,
             out_hbm, agg_hbm,
             rowidx_v, colidx_v, p_v, q_v, r_v, o_v, of_v, agg_sp,
             sem_in, sem_out):
    cid = lax.axis_index("c")
    sid = lax.axis_index("s")
    wid = cid * _NS + sid
    ebase = wid * _EPW

    pltpu.sync_copy(z_hbm.at[pl.ds(0, _RPS)], agg_sp.at[pl.ds(sid * _RPS, _RPS)])

    @pl.when(sid == 0)
    def _zero_tail():
        pltpu.sync_copy(z_hbm.at[pl.ds(0, _RTAIL)],
                        agg_sp.at[pl.ds(_NS * _RPS, _RTAIL)])

    pltpu.sync_copy(ei_hbm.at[0].at[wid], rowidx_v)
    pltpu.sync_copy(ei_hbm.at[1].at[wid], colidx_v)

    plsc.subcore_barrier()

    blk0 = wid * _NBLK

    def in_descs(g, slot, sem):
        ds = []
        for b in range(_G):
            i = g * _G + b
            ds.append(pltpu.make_async_copy(
                p_hbm.at[rowidx_v.at[i]], p_v.at[slot].at[b], sem))
            ds.append(pltpu.make_async_copy(
                q_hbm.at[colidx_v.at[i]], q_v.at[slot].at[b], sem))
        ds.append(pltpu.make_async_copy(
            r_hbm.at[pl.ds(blk0 + g * _G, _G)], r_v.at[slot], sem))
        return ds

    def fire_in(g, slot, sem):
        for d in in_descs(g, slot, sem):
            d.start()

    def drain_in(g, slot, sem):
        for d in in_descs(g, slot, sem):
            d.wait()

    def compute_out(g, slot):
        def body(jr, c):
            for b in range(_G):
                for cc in range(8):
                    j = jr * 8 + cc
                    o = jnp.maximum(
                        p_v[slot, b, j, :] + q_v[slot, b, j, :]
                        + r_v[slot, b, j, :], 0.0)
                    o_v[b, j, :] = o
                    of_v[b * (_BLK // 8) + jr, pl.ds(cc * _DE, _DE)] = o
            return c

        lax.fori_loop(0, _BLK // 8, body, 0)

        pltpu.sync_copy(of_v,
                        out_hbm.at[pl.ds((ebase + g * _GE) // 8, _GE // 8)])
        for b in range(_G):
            i = g * _G + b
            pltpu.sync_copy(o_v.at[b], agg_sp.at[colidx_v.at[i]], add=True)

    fire_in(0, 0, sem_in)

    def it(t, carry):
        g0 = 2 * t
        g1 = 2 * t + 1
        fire_in(g1, 1, sem_out)
        drain_in(g0, 0, sem_in)
        compute_out(g0, 0)
        fire_in(g1 + 1, 0, sem_in)
        drain_in(g1, 1, sem_out)
        compute_out(g1, 1)
        return carry

    lax.fori_loop(0, (_NG - 1) // 2, it, 0)
    drain_in(_NG - 1, 0, sem_in)
    compute_out(_NG - 1, 0)

    plsc.subcore_barrier()
    pltpu.sync_copy(
        agg_sp.at[pl.ds(sid * _RPS, _RPS)],
        agg_hbm.at[cid].at[pl.ds(sid * _RPS, _RPS)],
    )

    @pl.when(sid == 0)
    def _write_tail():
        pltpu.sync_copy(
            agg_sp.at[pl.ds(_NS * _RPS, _RTAIL)],
            agg_hbm.at[cid].at[pl.ds(_NS * _RPS, _RTAIL)],
        )


def _node_body(x_ref, a0_ref, a1_ref, wx_ref, wa_ref, b_ref, o_ref):
    acc = jnp.dot(x_ref[...], wx_ref[...], preferred_element_type=jnp.float32)
    acc += jnp.dot(a0_ref[...] + a1_ref[...], wa_ref[...],
                   preferred_element_type=jnp.float32)
    o_ref[...] = jnp.maximum(acc + b_ref[...], 0.0)


def kernel(x, edge_index, edge_attr, W_edge, b_edge, W_node, b_node):
    Ws = W_edge[:_D]
    Wd = W_edge[_D:2 * _D]
    Wa = W_edge[2 * _D:]
    W_sd = jnp.concatenate([Ws, Wd], axis=1)
    W_blk = jax.scipy.linalg.block_diag(*([Wa] * 8))
    b8 = jnp.tile(b_edge, 8).reshape(1, _D)

    ei = edge_index.reshape(2, _NW, _NBLK, _BLK)

    nb = 5
    P, Q = pl.pallas_call(
        _pq_body,
        grid=(nb,),
        in_specs=[
            pl.BlockSpec((_N // nb, _D), lambda i: (i, 0)),
            pl.BlockSpec((_D, 2 * _DE), lambda i: (0, 0)),
        ],
        out_specs=[
            pl.BlockSpec((_N // nb, _DE), lambda i: (i, 0)),
            pl.BlockSpec((_N // nb, _DE), lambda i: (i, 0)),
        ],
        out_shape=[jax.ShapeDtypeStruct((_N, _DE), jnp.float32)] * 2,
    )(x, W_sd)

    er, rb = _E // 8, 10
    R = pl.pallas_call(
        _r_body,
        grid=(rb,),
        in_specs=[
            pl.BlockSpec((er // rb, _D), lambda i: (i, 0)),
            pl.BlockSpec((_D, _D), lambda i: (0, 0)),
            pl.BlockSpec((1, _D), lambda i: (0, 0)),
        ],
        out_specs=pl.BlockSpec((er // rb, _D), lambda i: (i, 0)),
        out_shape=jax.ShapeDtypeStruct((er, _D), jnp.float32),
    )(edge_attr.reshape(er, _D), W_blk, b8)

    zeros = jnp.zeros((_RPS, _DE), jnp.float32)
    edge_attr_new, agg2 = _sc_edge(
        P, Q, R.reshape(_NW * _NBLK, _BLK, _DE), ei, zeros)
    edge_attr_new = edge_attr_new.reshape(_E, _DE)

    xb = 5
    x_new = pl.pallas_call(
        _node_body,
        grid=(xb,),
        in_specs=[
            pl.BlockSpec((_N // xb, _D), lambda i: (i, 0)),
            pl.BlockSpec((_N // xb, _DE), lambda i: (i, 0)),
            pl.BlockSpec((_N // xb, _DE), lambda i: (i, 0)),
            pl.BlockSpec((_D, _D), lambda i: (0, 0)),
            pl.BlockSpec((_DE, _D), lambda i: (0, 0)),
            pl.BlockSpec((1, _D), lambda i: (0, 0)),
        ],
        out_specs=pl.BlockSpec((_N // xb, _D), lambda i: (i, 0)),
        out_shape=jax.ShapeDtypeStruct((_N, _D), jnp.float32),
    )(x, agg2[0], agg2[1], W_node[:_D], W_node[_D:], b_node.reshape(1, _D))

    return (x_new, edge_attr_new)
